# baseline (speedup 1.0000x reference)
import functools

import jax
import jax.numpy as jnp
from jax import lax
from jax.experimental import pallas as pl
from jax.experimental.pallas import tpu as pltpu

B = 16
H = 4
TB = 4096

_LOG2E = 1.4426950408889634
_LN2 = 0.6931471805599453


def _softplus(x):
    t = jnp.exp2(jnp.abs(x) * -_LOG2E)
    return jnp.maximum(x, 0.0) + _LN2 * jnp.log2(1.0 + t)


def _body(cu, x0, x1, w10, b10, w20, b20, w11, b11, w21, b21, wb, bb_, wo,
          bo_, out, facc0, facc1, wacc, *, nt, d, hid):
    b10r = b10[...].reshape(1, d)
    b11r = b11[...].reshape(1, d)
    b20r = b20[...].reshape(1, H)
    b21r = b21[...].reshape(1, H)
    bbr = bb_[...].reshape(1, hid)
    bor = bo_[...].reshape(1, -1)
    i = pl.program_id(0)

    @pl.when(i == 0)
    def _init():
        facc0[...] = jnp.zeros_like(facc0)
        facc1[...] = jnp.zeros_like(facc1)
        wacc[...] = jnp.zeros_like(wacc)

    zcol = jnp.zeros((d, B * H), jnp.float32)
    w2e0 = jnp.concatenate(
        [jnp.broadcast_to(w20[:, hh:hh + 1], (d, B)) for hh in range(H)]
        + [zcol], axis=1)
    w2e1 = jnp.concatenate(
        [zcol]
        + [jnp.broadcast_to(w21[:, hh:hh + 1], (d, B)) for hh in range(H)],
        axis=1)
    b2both = jnp.concatenate(
        [jnp.broadcast_to(b20r[:, hh:hh + 1], (1, B)) for hh in range(H)]
        + [jnp.broadcast_to(b21r[:, hh:hh + 1], (1, B)) for hh in range(H)],
        axis=1)

    bagv = lax.broadcasted_iota(jnp.int32, (1, 2 * B * H), 1) % B
    lo = jnp.zeros((1, 2 * B * H), jnp.int32)
    hi = jnp.zeros((1, 2 * B * H), jnp.int32)
    for b in range(B):
        lo = jnp.where(bagv == b, cu[b], lo)
        hi = jnp.where(bagv == b, cu[b + 1], hi)

    g = lax.broadcasted_iota(jnp.int32, (TB, 1), 0) + i * TB
    mask = (g >= lo) & (g < hi)
    ones = jnp.ones((TB, 1), jnp.float32)
    cdims = (((0,), (0,)), ((), ()))

    x0v = x0[...]
    x1v = x1[...]
    h0 = _softplus(
        jnp.dot(x0v, w10[...], preferred_element_type=jnp.float32) + b10r)
    h1 = _softplus(
        jnp.dot(x1v, w11[...], preferred_element_type=jnp.float32) + b11r)
    apre = (jnp.dot(h0, w2e0, preferred_element_type=jnp.float32)
            + jnp.dot(h1, w2e1, preferred_element_type=jnp.float32) + b2both)
    ap = jnp.where(mask, _softplus(apre), 0.0)
    facc0[...] += lax.dot_general(ap[:, :B * H], x0v, cdims,
                                  preferred_element_type=jnp.float32)
    facc1[...] += lax.dot_general(ap[:, B * H:], x1v, cdims,
                                  preferred_element_type=jnp.float32)
    wacc[...] += lax.dot_general(ap, ones, cdims,
                                 preferred_element_type=jnp.float32)

    @pl.when(i == nt - 1)
    def _finish():
        hb = jnp.zeros((B, hid), jnp.float32) + bbr
        w = wacc[...]
        for t, fa in enumerate((facc0, facc1)):
            f = fa[...]
            for hh in range(H):
                fs = f[hh * B:(hh + 1) * B, :]
                den = w[t * B * H + hh * B:t * B * H + (hh + 1) * B, :]
                favg = jnp.where(den != 0.0,
                                 fs / jnp.where(den == 0.0, 1.0, den), 0.0)
                favg = jnp.where(jnp.isnan(favg), 1e-5, favg)
                row0 = t * H * d + hh * d
                hb += jnp.dot(favg, wb[row0:row0 + d, :],
                              preferred_element_type=jnp.float32)
        hb = _softplus(hb)
        logits = jnp.dot(hb, wo[...],
                         preferred_element_type=jnp.float32) + bor
        m = jnp.max(logits, axis=-1, keepdims=True)
        e = jnp.exp(logits - m)
        out[...] = e / jnp.sum(e, axis=-1, keepdims=True)


def kernel(flat_x_tube0, flat_x_tube1, W1_0, b1_0, W2_0, b2_0,
           W1_1, b1_1, W2_1, b2_1, Wb, bb, Wo, bo, cu_seqlens):
    T, D = flat_x_tube0.shape
    HID = Wb.shape[1]
    C = Wo.shape[1]
    nt = T // TB

    whole = lambda shape: pl.BlockSpec(shape, lambda i: tuple(0 for _ in shape))

    body = functools.partial(_body, nt=nt, d=D, hid=HID)
    return pl.pallas_call(
        body,
        grid=(nt,),
        in_specs=[
            pl.BlockSpec(memory_space=pltpu.SMEM),
            pl.BlockSpec((TB, D), lambda i: (i, 0)),
            pl.BlockSpec((TB, D), lambda i: (i, 0)),
            whole((D, D)), whole((D,)), whole((D, H)), whole((H,)),
            whole((D, D)), whole((D,)), whole((D, H)), whole((H,)),
            whole((2 * H * D, HID)), whole((HID,)),
            whole((HID, C)), whole((C,)),
        ],
        out_specs=pl.BlockSpec((B, C), lambda i: (0, 0)),
        out_shape=jax.ShapeDtypeStruct((B, C), jnp.float32),
        scratch_shapes=[
            pltpu.VMEM((B * H, D), jnp.float32),
            pltpu.VMEM((B * H, D), jnp.float32),
            pltpu.VMEM((2 * B * H, 1), jnp.float32),
        ],
    )(cu_seqlens, flat_x_tube0, flat_x_tube1,
      W1_0, b1_0, W2_0, b2_0,
      W1_1, b1_1, W2_1, b2_1,
      Wb, bb, Wo, bo)


# speedup vs baseline: 1.0332x; 1.0332x over previous
import functools

import jax
import jax.numpy as jnp
from jax import lax
from jax.experimental import pallas as pl
from jax.experimental.pallas import tpu as pltpu

B = 16
H = 4
TB = 4096

_LOG2E = 1.4426950408889634
_LN2 = 0.6931471805599453


def _softplus(x):
    t = jnp.exp2(jnp.abs(x) * -_LOG2E)
    return jnp.maximum(x, 0.0) + _LN2 * jnp.log2(1.0 + t)


def _softplus_l2(y):
    t = jnp.exp2(-jnp.abs(y))
    return jnp.maximum(y, 0.0) + jnp.log2(1.0 + t)


def _body(cu, x0, x1, w10, b10, w20, b20, w11, b11, w21, b21, wb, bb_, wo,
          bo_, out, facc0, facc1, wacc, *, nt, d, hid):
    b10r = b10[...].reshape(1, d)
    b11r = b11[...].reshape(1, d)
    b20r = b20[...].reshape(1, H)
    b21r = b21[...].reshape(1, H)
    bbr = bb_[...].reshape(1, hid)
    bor = bo_[...].reshape(1, -1)
    i = pl.program_id(0)

    @pl.when(i == 0)
    def _init():
        facc0[...] = jnp.zeros_like(facc0)
        facc1[...] = jnp.zeros_like(facc1)
        wacc[...] = jnp.zeros_like(wacc)

    zcol = jnp.zeros((d, B * H), jnp.float32)
    w2e0 = jnp.concatenate(
        [jnp.broadcast_to(w20[:, hh:hh + 1], (d, B)) for hh in range(H)]
        + [zcol], axis=1)
    w2e1 = jnp.concatenate(
        [zcol]
        + [jnp.broadcast_to(w21[:, hh:hh + 1], (d, B)) for hh in range(H)],
        axis=1)
    b2both = jnp.concatenate(
        [jnp.broadcast_to(b20r[:, hh:hh + 1], (1, B)) for hh in range(H)]
        + [jnp.broadcast_to(b21r[:, hh:hh + 1], (1, B)) for hh in range(H)],
        axis=1)

    bagv = lax.broadcasted_iota(jnp.int32, (1, 2 * B * H), 1) % B
    lo = jnp.zeros((1, 2 * B * H), jnp.int32)
    hi = jnp.zeros((1, 2 * B * H), jnp.int32)
    for b in range(B):
        lo = jnp.where(bagv == b, cu[b], lo)
        hi = jnp.where(bagv == b, cu[b + 1], hi)

    g = lax.broadcasted_iota(jnp.int32, (TB, 1), 0) + i * TB
    mask = (g >= lo) & (g < hi)
    ones = jnp.ones((TB, 1), jnp.float32)
    cdims = (((0,), (0,)), ((), ()))

    x0v = x0[...]
    x1v = x1[...]
    h0 = _softplus_l2(
        jnp.dot(x0v, w10[...] * _LOG2E,
                preferred_element_type=jnp.float32) + b10r * _LOG2E)
    h1 = _softplus_l2(
        jnp.dot(x1v, w11[...] * _LOG2E,
                preferred_element_type=jnp.float32) + b11r * _LOG2E)
    apre = (jnp.dot(h0, w2e0, preferred_element_type=jnp.float32)
            + jnp.dot(h1, w2e1, preferred_element_type=jnp.float32)
            + b2both * _LOG2E)
    ap = jnp.where(mask, _softplus_l2(apre), 0.0)
    facc0[...] += lax.dot_general(ap[:, :B * H], x0v, cdims,
                                  preferred_element_type=jnp.float32)
    facc1[...] += lax.dot_general(ap[:, B * H:], x1v, cdims,
                                  preferred_element_type=jnp.float32)
    wacc[...] += lax.dot_general(ap, ones, cdims,
                                 preferred_element_type=jnp.float32)

    @pl.when(i == nt - 1)
    def _finish():
        hb = jnp.zeros((B, hid), jnp.float32) + bbr
        w = wacc[...]
        for t, fa in enumerate((facc0, facc1)):
            f = fa[...]
            for hh in range(H):
                fs = f[hh * B:(hh + 1) * B, :]
                den = w[t * B * H + hh * B:t * B * H + (hh + 1) * B, :]
                favg = jnp.where(den != 0.0,
                                 fs / jnp.where(den == 0.0, 1.0, den), 0.0)
                favg = jnp.where(jnp.isnan(favg), 1e-5, favg)
                row0 = t * H * d + hh * d
                hb += jnp.dot(favg, wb[row0:row0 + d, :],
                              preferred_element_type=jnp.float32)
        hb = _softplus(hb)
        logits = jnp.dot(hb, wo[...],
                         preferred_element_type=jnp.float32) + bor
        m = jnp.max(logits, axis=-1, keepdims=True)
        e = jnp.exp(logits - m)
        out[...] = e / jnp.sum(e, axis=-1, keepdims=True)


def kernel(flat_x_tube0, flat_x_tube1, W1_0, b1_0, W2_0, b2_0,
           W1_1, b1_1, W2_1, b2_1, Wb, bb, Wo, bo, cu_seqlens):
    T, D = flat_x_tube0.shape
    HID = Wb.shape[1]
    C = Wo.shape[1]
    nt = T // TB

    whole = lambda shape: pl.BlockSpec(shape, lambda i: tuple(0 for _ in shape))

    body = functools.partial(_body, nt=nt, d=D, hid=HID)
    return pl.pallas_call(
        body,
        grid=(nt,),
        in_specs=[
            pl.BlockSpec(memory_space=pltpu.SMEM),
            pl.BlockSpec((TB, D), lambda i: (i, 0)),
            pl.BlockSpec((TB, D), lambda i: (i, 0)),
            whole((D, D)), whole((D,)), whole((D, H)), whole((H,)),
            whole((D, D)), whole((D,)), whole((D, H)), whole((H,)),
            whole((2 * H * D, HID)), whole((HID,)),
            whole((HID, C)), whole((C,)),
        ],
        out_specs=pl.BlockSpec((B, C), lambda i: (0, 0)),
        out_shape=jax.ShapeDtypeStruct((B, C), jnp.float32),
        scratch_shapes=[
            pltpu.VMEM((B * H, D), jnp.float32),
            pltpu.VMEM((B * H, D), jnp.float32),
            pltpu.VMEM((2 * B * H, 1), jnp.float32),
        ],
    )(cu_seqlens, flat_x_tube0, flat_x_tube1,
      W1_0, b1_0, W2_0, b2_0,
      W1_1, b1_1, W2_1, b2_1,
      Wb, bb, Wo, bo)
